# Initial kernel scaffold; baseline (speedup 1.0000x reference)
import jax
import jax.numpy as jnp
from jax.experimental import pallas as pl


def kernel(x, pre_bias, W_enc, b_enc, W_dec):
    raise NotImplementedError("write your pallas kernel here")


# speedup vs baseline: 3.0529x; 3.0529x over previous
import functools

import jax
import jax.numpy as jnp
from jax import lax
from jax.experimental import pallas as pl

B = 128
D = 768
F = 65536
K = 64

FC = 4096
RB = 8


def _enc_body(x_ref, pb_ref, w_ref, b_ref, z_ref):
    xc = x_ref[...] - pb_ref[...][None, :]
    z = lax.dot_general(xc, w_ref[...], (((1,), (1,)), ((), ())),
                        precision=lax.Precision.DEFAULT,
                        preferred_element_type=jnp.float32)
    z_ref[...] = z + b_ref[...][0][None, :]


def _encode(x, pre_bias, W_enc, b_enc):
    grid = F // FC
    return pl.pallas_call(
        _enc_body,
        grid=(grid,),
        in_specs=[
            pl.BlockSpec((B, D), lambda i: (0, 0)),
            pl.BlockSpec((D,), lambda i: (0,)),
            pl.BlockSpec((FC, D), lambda i: (i, 0)),
            pl.BlockSpec((1, FC), lambda i: (0, i)),
        ],
        out_specs=pl.BlockSpec((B, FC), lambda i: (0, i)),
        out_shape=jax.ShapeDtypeStruct((B, F), jnp.float32),
    )(x, pre_bias, W_enc, b_enc.reshape(1, F))


def _topk_body(z_ref, sparse_ref):
    z = z_ref[...]
    b = lax.bitcast_convert_type(z, jnp.int32)
    m = (b >> 31) | jnp.int32(-2147483648)
    u = lax.bitcast_convert_type(b ^ m, jnp.uint32)

    def bit_step(i, t):
        cand = t | (jnp.uint32(1) << (jnp.uint32(31) - i.astype(jnp.uint32)))
        cnt = jnp.sum((u >= cand[:, None]).astype(jnp.float32), axis=1)
        return jnp.where(cnt >= K, cand, t)

    T = lax.fori_loop(0, 32, bit_step, jnp.zeros((RB,), jnp.uint32))
    Tb = T[:, None]

    gt = u > Tb
    eq = u == Tb
    n_gt = jnp.sum(gt.astype(jnp.float32), axis=1, keepdims=True)
    need_eq = jnp.float32(K) - n_gt

    eqf = eq.astype(jnp.float32).reshape(RB * 512, 128)
    r_i = lax.broadcasted_iota(jnp.int32, (128, 128), 0)
    c_i = lax.broadcasted_iota(jnp.int32, (128, 128), 1)
    incl = (r_i <= c_i).astype(jnp.float32)
    within = lax.dot_general(eqf, incl, (((1,), (0,)), ((), ())),
                             precision=lax.Precision.HIGHEST,
                             preferred_element_type=jnp.float32)
    chunk_tot = jnp.sum(eqf, axis=1).reshape(RB, 512)
    r2 = lax.broadcasted_iota(jnp.int32, (512, 512), 0)
    c2 = lax.broadcasted_iota(jnp.int32, (512, 512), 1)
    strict = (r2 < c2).astype(jnp.float32)
    offs = lax.dot_general(chunk_tot, strict, (((1,), (0,)), ((), ())),
                           precision=lax.Precision.HIGHEST,
                           preferred_element_type=jnp.float32)
    rank = (within.reshape(RB, 512, 128)
            + offs[:, :, None]).reshape(RB, F)

    sel = gt | (eq & (rank <= need_eq))
    sparse_ref[...] = jnp.where(sel, jnp.maximum(z, 0.0), 0.0)


def _topk_sparse(z):
    grid = B // RB
    return pl.pallas_call(
        _topk_body,
        grid=(grid,),
        in_specs=[pl.BlockSpec((RB, F), lambda i: (i, 0))],
        out_specs=pl.BlockSpec((RB, F), lambda i: (i, 0)),
        out_shape=jax.ShapeDtypeStruct((B, F), jnp.float32),
    )(z)


def _dec_body(s_ref, w_ref, pb_ref, out_ref):
    i = pl.program_id(0)
    part = lax.dot_general(s_ref[...], w_ref[...], (((1,), (1,)), ((), ())),
                           precision=lax.Precision.DEFAULT,
                           preferred_element_type=jnp.float32)

    @pl.when(i == 0)
    def _init():
        out_ref[...] = part + pb_ref[...][None, :]

    @pl.when(i != 0)
    def _acc():
        out_ref[...] += part


def _decode(sparse, W_dec, pre_bias):
    grid = F // FC
    return pl.pallas_call(
        _dec_body,
        grid=(grid,),
        in_specs=[
            pl.BlockSpec((B, FC), lambda i: (0, i)),
            pl.BlockSpec((D, FC), lambda i: (0, i)),
            pl.BlockSpec((D,), lambda i: (0,)),
        ],
        out_specs=pl.BlockSpec((B, D), lambda i: (0, 0)),
        out_shape=jax.ShapeDtypeStruct((B, D), jnp.float32),
    )(sparse, W_dec, pre_bias)


def kernel(x, pre_bias, W_enc, b_enc, W_dec):
    z = _encode(x, pre_bias, W_enc, b_enc)
    sparse = _topk_sparse(z)
    x_hat = _decode(sparse, W_dec, pre_bias)
    return (x_hat, sparse)
